# baseline (speedup 1.0000x reference)
import functools

import jax
import jax.numpy as jnp
from jax import lax
from jax.experimental import pallas as pl
from jax.experimental.pallas import tpu as pltpu
from jax.experimental.pallas import tpu_sc as plsc

E = 8
TOPK = 2
BM = 128

NC = 2
NS = 16
NW = NC * NS


def _router_body(x_ref, gw_ref, pos_ref, wbc_ref, bexp_ref, xbf_ref):
    T = x_ref.shape[0]
    nbpad = bexp_ref.shape[1]
    f32 = jnp.float32

    x = x_ref[...]
    xbf_ref[...] = x.astype(jnp.bfloat16)
    gw = gw_ref[...]
    logits = lax.dot_general(x, gw, (((1,), (1,)), ((), ())),
                             preferred_element_type=f32)

    ut8 = (lax.broadcasted_iota(jnp.int32, (E, E), 0)
           <= lax.broadcasted_iota(jnp.int32, (E, E), 1)).astype(f32)
    l1 = jnp.max(logits, axis=1, keepdims=True)
    oh1 = (logits == l1).astype(f32)
    oh1 = oh1 * (lax.dot_general(oh1, ut8, (((1,), (0,)), ((), ())),
                                 preferred_element_type=f32) == 1.0)
    neg = jnp.float32(-1e30)
    masked = jnp.where(oh1 > 0, neg, logits)
    l2 = jnp.max(masked, axis=1, keepdims=True)
    oh2 = (masked == l2).astype(f32)
    oh2 = oh2 * (lax.dot_general(oh2, ut8, (((1,), (0,)), ((), ())),
                                 preferred_element_type=f32) == 1.0)

    w0 = jax.nn.sigmoid(l1 - l2)
    w1 = jax.nn.sigmoid(l2 - l1)
    wbc_ref[0] = jnp.broadcast_to(w0, (T, 16))
    wbc_ref[1] = jnp.broadcast_to(w1, (T, 16))

    lt = (lax.broadcasted_iota(jnp.int32, (T, T), 1)
          <= lax.broadcasted_iota(jnp.int32, (T, T), 0)).astype(f32)
    c1 = lax.dot_general(lt, oh1, (((1,), (0,)), ((), ())),
                         preferred_element_type=f32)
    c2 = lax.dot_general(lt, oh2, (((1,), (0,)), ((), ())),
                         preferred_element_type=f32)
    count1 = jnp.sum(oh1, axis=0, keepdims=True)
    count2 = jnp.sum(oh2, axis=0, keepdims=True)
    counts = count1 + count2

    bmf = jnp.float32(BM)
    padded = jnp.floor((counts + (bmf - 1.0)) / bmf) * bmf
    slt8 = (lax.broadcasted_iota(jnp.int32, (E, E), 0)
            < lax.broadcasted_iota(jnp.int32, (E, E), 1)).astype(f32)
    off = lax.dot_general(padded, slt8, (((1,), (0,)), ((), ())),
                          precision=lax.Precision.HIGHEST,
                          preferred_element_type=f32)

    pos0 = jnp.sum(oh1 * (off + c1 - 1.0), axis=1)
    pos1 = jnp.sum(oh2 * (off + count1 + c2 - 1.0), axis=1)
    pos_ref[0, :] = pos0.astype(jnp.int32)
    pos_ref[1, :] = pos1.astype(jnp.int32)

    i8 = (lax.broadcasted_iota(jnp.int32, (E, E), 0)
          == lax.broadcasted_iota(jnp.int32, (E, E), 1)).astype(f32)
    ones81 = jnp.ones((E, 1), f32)
    off_col = lax.dot_general(i8 * off, ones81, (((1,), (0,)), ((), ())),
                              precision=lax.Precision.HIGHEST,
                              preferred_element_type=f32)
    cnt_col = lax.dot_general(i8 * counts, ones81, (((1,), (0,)), ((), ())),
                              precision=lax.Precision.HIGHEST,
                              preferred_element_type=f32)
    bs_col = jnp.floor(off_col / bmf)
    nb_col = jnp.floor((cnt_col + (bmf - 1.0)) / bmf)
    bi = lax.broadcasted_iota(jnp.int32, (E, nbpad), 1).astype(f32)
    belongs = ((bi >= bs_col) & (bi < bs_col + nb_col)).astype(f32)
    erow = lax.broadcasted_iota(jnp.int32, (E, nbpad), 0).astype(f32)
    bexp = jnp.sum(erow * belongs, axis=0, keepdims=True)
    assigned = jnp.sum(belongs, axis=0, keepdims=True)
    bexp = jnp.where(assigned > 0, bexp, jnp.float32(E - 1))
    bexp_ref[0, :] = bexp.astype(jnp.int32)[0]

    nonempty = (nb_col > 0).astype(f32)
    isf = jnp.sum(((bi == bs_col).astype(f32)) * nonempty, axis=0,
                  keepdims=True)
    ut128 = (lax.broadcasted_iota(jnp.int32, (nbpad, nbpad), 0)
             <= lax.broadcasted_iota(jnp.int32, (nbpad, nbpad), 1)).astype(f32)
    cum = lax.dot_general(isf, ut128, (((1,), (0,)), ((), ())),
                          precision=lax.Precision.HIGHEST,
                          preferred_element_type=f32)
    parity = (cum.astype(jnp.int32) - 1) & 1
    cand = jnp.where((bs_col > bi) & (nb_col > 0), erow, jnp.float32(E))
    nexte = jnp.min(cand, axis=0, keepdims=True)
    bexp_ref[1, :] = isf.astype(jnp.int32)[0]
    bexp_ref[2, :] = parity[0]
    bexp_ref[3, :] = nexte.astype(jnp.int32)[0]


def _dispatch_body(x_hbm, pos_hbm, xs_hbm, rows0_v, rows1_v, idx_v,
                   semL0, semL1, semS):
    T = x_hbm.shape[0]
    ch = rows0_v.shape[0]
    tpw = T // NW
    wid = lax.axis_index("s") * NC + lax.axis_index("c")
    base = wid * tpw
    nk = tpw // ch
    rows = (rows0_v, rows1_v)
    semsl = (semL0, semL1)
    pltpu.sync_copy(pos_hbm.at[pl.ds(base, tpw)], idx_v.at[0])
    pltpu.sync_copy(pos_hbm.at[pl.ds(T + base, tpw)], idx_v.at[1])
    loads = [None] * nk
    loads[0] = pltpu.async_copy(x_hbm.at[pl.ds(base, ch)], rows[0], semsl[0])
    for k in range(nk):
        cur = k % 2
        if k + 1 < nk:
            loads[k + 1] = pltpu.async_copy(
                x_hbm.at[pl.ds(base + (k + 1) * ch, ch)],
                rows[(k + 1) % 2], semsl[(k + 1) % 2])
        loads[k].wait()
        i0 = idx_v[0, pl.ds(k * ch, ch)]
        i1 = idx_v[1, pl.ds(k * ch, ch)]
        d0 = pltpu.async_copy(rows[cur], xs_hbm.at[i0], semS)
        d1 = pltpu.async_copy(rows[cur], xs_hbm.at[i1], semS)
        d0.wait()
        d1.wait()


def _combine_body(ys_hbm, pos_hbm, wbc_hbm, out_hbm,
                  g0a_v, g0b_v, g1a_v, g1b_v, oa_v, ob_v,
                  i0a_v, i0b_v, i1a_v, i1b_v, wb_v,
                  semGa, semGb, semOa, semOb):
    T, H = out_hbm.shape
    ct = g0a_v.shape[0]
    tpw = T // NW
    wid = lax.axis_index("s") * NC + lax.axis_index("c")
    base = wid * tpw
    nk = tpw // ct
    g0 = (g0a_v, g0b_v)
    g1 = (g1a_v, g1b_v)
    ov = (oa_v, ob_v)
    pltpu.sync_copy(wbc_hbm.at[pl.ds(base, tpw), :], wb_v.at[0])
    pltpu.sync_copy(wbc_hbm.at[pl.ds(T + base, tpw), :], wb_v.at[1])

    semg = (semGa, semGb)
    semo = (semOa, semOb)
    i0r = (i0a_v, i0b_v)
    i1r = (i1a_v, i1b_v)

    def gathers(k):
        s = k % 2
        pltpu.sync_copy(pos_hbm.at[pl.ds(base + k * ct, ct)], i0r[s])
        pltpu.sync_copy(pos_hbm.at[pl.ds(T + base + k * ct, ct)], i1r[s])
        return (pltpu.async_copy(ys_hbm.at[i0r[s]], g0[s], semg[s]),
                pltpu.async_copy(ys_hbm.at[i1r[s]], g1[s], semg[s]))

    pend = gathers(0)
    stores = [None] * nk
    for k in range(nk):
        s = k % 2
        nxt = None
        if k + 1 < nk:
            nxt = gathers(k + 1)
        pend[0].wait()
        pend[1].wait()
        pend = nxt
        if k >= 2:
            stores[k - 2].wait()
        for i in range(ct):
            wb0 = wb_v[0, k * ct + i]
            wb1 = wb_v[1, k * ct + i]

            @plsc.parallel_loop(0, H, step=16, unroll=8)
            def _mul(c):
                sl = pl.ds(c, 16)
                ov[s][i, sl] = g0[s][i, sl] * wb0 + g1[s][i, sl] * wb1
        stores[k] = pltpu.async_copy(
            ov[s], out_hbm.at[pl.ds(base + k * ct, ct), :], semo[s])
    stores[nk - 2].wait()
    stores[nk - 1].wait()


def _mm_body(meta_ref, xs_ref, gu_hbm, dn_hbm, ys_ref,
             gu0, gu1, dn0, dn1, sg0, sg1, sd0, sd1):
    nbpad = 128
    I = dn0.shape[0]
    b = pl.program_id(0)
    e = meta_ref[b]
    isf = meta_ref[nbpad + b]
    p = meta_ref[2 * nbpad + b]
    ne = meta_ref[3 * nbpad + b]
    gubuf = (gu0, gu1)
    dnbuf = (dn0, dn1)
    semg = (sg0, sg1)
    semd = (sd0, sd1)

    def issue(slot, ex):
        pltpu.make_async_copy(gu_hbm.at[ex], gubuf[slot], semg[slot]).start()
        pltpu.make_async_copy(dn_hbm.at[ex], dnbuf[slot], semd[slot]).start()

    def waitw(slot, ex):
        pltpu.make_async_copy(gu_hbm.at[ex], gubuf[slot], semg[slot]).wait()
        pltpu.make_async_copy(dn_hbm.at[ex], dnbuf[slot], semd[slot]).wait()

    @pl.when(b == 0)
    def _cold():
        issue(0, e)

    for slot in range(2):
        @pl.when((isf == 1) & (p == slot))
        def _first(slot=slot):
            waitw(slot, e)

        @pl.when((isf == 1) & (ne < E) & (p == slot))
        def _issue_next(slot=slot):
            issue(1 - slot, ne)

        @pl.when(p == slot)
        def _compute(slot=slot):
            xb = xs_ref[...].astype(jnp.float32)
            h1 = lax.dot_general(xb, gubuf[slot][...],
                                 (((1,), (0,)), ((), ())),
                                 preferred_element_type=jnp.float32)
            g = h1[:, :I]
            u = h1[:, I:]
            act = g * jax.nn.sigmoid(g) * u
            ys_ref[...] = lax.dot_general(act, dnbuf[slot][...],
                                          (((1,), (0,)), ((), ())),
                                          preferred_element_type=jnp.float32)


def kernel(hidden_states, gate_weight, gate_up_proj, down_proj):
    b, s, h = hidden_states.shape
    e, _, i2 = gate_up_proj.shape
    i = i2 // 2
    T = b * s
    nb = (2 * T) // BM + e - 1
    pad = nb * BM
    nbpad = 128

    x = hidden_states.reshape(T, h)

    pos2, wbc3, meta_rows, xbf = pl.pallas_call(
        _router_body,
        out_shape=(
            jax.ShapeDtypeStruct((2, T), jnp.int32),
            jax.ShapeDtypeStruct((2, T, 16), jnp.float32),
            jax.ShapeDtypeStruct((4, nbpad), jnp.int32),
            jax.ShapeDtypeStruct((T, h), jnp.bfloat16),
        ),
    )(x, gate_weight)
    pos = pos2.reshape(2 * T)
    wbc = wbc3.reshape(2 * T, 16)
    meta = meta_rows.reshape(4 * nbpad)
    hw = h // 2
    xw = lax.bitcast_convert_type(xbf.reshape(T, hw, 2), jnp.int32)

    ch = 16
    tpw = T // NW
    mesh = plsc.VectorSubcoreMesh(core_axis_name="c", subcore_axis_name="s")
    xs_w = pl.kernel(
        _dispatch_body,
        out_type=jax.ShapeDtypeStruct((pad, hw), jnp.int32),
        mesh=mesh,
        scratch_types=[
            pltpu.VMEM((ch, hw), jnp.int32),
            pltpu.VMEM((ch, hw), jnp.int32),
            pltpu.VMEM((2, tpw), jnp.int32),
            pltpu.SemaphoreType.DMA,
            pltpu.SemaphoreType.DMA,
            pltpu.SemaphoreType.DMA,
        ],
    )(xw, pos)
    xs = lax.bitcast_convert_type(xs_w, jnp.bfloat16).reshape(pad, h)

    grid_spec = pltpu.PrefetchScalarGridSpec(
        num_scalar_prefetch=1,
        grid=(nb,),
        in_specs=[
            pl.BlockSpec((BM, h), lambda b_, be: (b_, 0)),
            pl.BlockSpec(memory_space=pl.ANY),
            pl.BlockSpec(memory_space=pl.ANY),
        ],
        out_specs=pl.BlockSpec((BM, h), lambda b_, be: (b_, 0)),
        scratch_shapes=[
            pltpu.VMEM((h, i2), jnp.float32),
            pltpu.VMEM((h, i2), jnp.float32),
            pltpu.VMEM((i, h), jnp.float32),
            pltpu.VMEM((i, h), jnp.float32),
            pltpu.SemaphoreType.DMA,
            pltpu.SemaphoreType.DMA,
            pltpu.SemaphoreType.DMA,
            pltpu.SemaphoreType.DMA,
        ],
    )
    ys = pl.pallas_call(
        _mm_body,
        grid_spec=grid_spec,
        out_shape=jax.ShapeDtypeStruct((pad, h), jnp.float32),
    )(meta, xs, gate_up_proj, down_proj)

    ct = 8
    out = pl.kernel(
        _combine_body,
        out_type=jax.ShapeDtypeStruct((T, h), jnp.float32),
        mesh=mesh,
        scratch_types=[
            pltpu.VMEM((ct, h), jnp.float32),
            pltpu.VMEM((ct, h), jnp.float32),
            pltpu.VMEM((ct, h), jnp.float32),
            pltpu.VMEM((ct, h), jnp.float32),
            pltpu.VMEM((ct, h), jnp.float32),
            pltpu.VMEM((ct, h), jnp.float32),
            pltpu.VMEM((ct,), jnp.int32),
            pltpu.VMEM((ct,), jnp.int32),
            pltpu.VMEM((ct,), jnp.int32),
            pltpu.VMEM((ct,), jnp.int32),
            pltpu.VMEM((2, tpw, 16), jnp.float32),
            pltpu.SemaphoreType.DMA,
            pltpu.SemaphoreType.DMA,
            pltpu.SemaphoreType.DMA,
            pltpu.SemaphoreType.DMA,
        ],
    )(ys, pos, wbc)

    return out.reshape(b, s, h)


# speedup vs baseline: 2.5332x; 2.5332x over previous
import functools

import jax
import jax.numpy as jnp
from jax import lax
from jax.experimental import pallas as pl
from jax.experimental.pallas import tpu as pltpu
from jax.experimental.pallas import tpu_sc as plsc

E = 8
TOPK = 2
BM = 128

NC = 2
NS = 16
NW = NC * NS


def _router_body(x_ref, gw_ref, pos_ref, wbc_ref, bexp_ref):
    T = x_ref.shape[0]
    nbpad = bexp_ref.shape[1]
    f32 = jnp.float32

    x = x_ref[...]
    gw = gw_ref[...]
    logits = lax.dot_general(x, gw, (((1,), (1,)), ((), ())),
                             preferred_element_type=f32)

    ut8 = (lax.broadcasted_iota(jnp.int32, (E, E), 0)
           <= lax.broadcasted_iota(jnp.int32, (E, E), 1)).astype(f32)
    l1 = jnp.max(logits, axis=1, keepdims=True)
    oh1 = (logits == l1).astype(f32)
    oh1 = oh1 * (lax.dot_general(oh1, ut8, (((1,), (0,)), ((), ())),
                                 preferred_element_type=f32) == 1.0)
    neg = jnp.float32(-1e30)
    masked = jnp.where(oh1 > 0, neg, logits)
    l2 = jnp.max(masked, axis=1, keepdims=True)
    oh2 = (masked == l2).astype(f32)
    oh2 = oh2 * (lax.dot_general(oh2, ut8, (((1,), (0,)), ((), ())),
                                 preferred_element_type=f32) == 1.0)

    w0 = jax.nn.sigmoid(l1 - l2)
    w1 = jax.nn.sigmoid(l2 - l1)
    wbc_ref[0] = jnp.broadcast_to(w0, (T, 16))
    wbc_ref[1] = jnp.broadcast_to(w1, (T, 16))

    lt = (lax.broadcasted_iota(jnp.int32, (T, T), 1)
          <= lax.broadcasted_iota(jnp.int32, (T, T), 0)).astype(f32)
    c1 = lax.dot_general(lt, oh1, (((1,), (0,)), ((), ())),
                         preferred_element_type=f32)
    c2 = lax.dot_general(lt, oh2, (((1,), (0,)), ((), ())),
                         preferred_element_type=f32)
    count1 = jnp.sum(oh1, axis=0, keepdims=True)
    count2 = jnp.sum(oh2, axis=0, keepdims=True)
    counts = count1 + count2

    bmf = jnp.float32(BM)
    padded = jnp.floor((counts + (bmf - 1.0)) / bmf) * bmf
    slt8 = (lax.broadcasted_iota(jnp.int32, (E, E), 0)
            < lax.broadcasted_iota(jnp.int32, (E, E), 1)).astype(f32)
    off = lax.dot_general(padded, slt8, (((1,), (0,)), ((), ())),
                          precision=lax.Precision.HIGHEST,
                          preferred_element_type=f32)

    pos0 = jnp.sum(oh1 * (off + c1 - 1.0), axis=1)
    pos1 = jnp.sum(oh2 * (off + count1 + c2 - 1.0), axis=1)
    pos_ref[0, :] = pos0.astype(jnp.int32)
    pos_ref[1, :] = pos1.astype(jnp.int32)

    i8 = (lax.broadcasted_iota(jnp.int32, (E, E), 0)
          == lax.broadcasted_iota(jnp.int32, (E, E), 1)).astype(f32)
    ones81 = jnp.ones((E, 1), f32)
    off_col = lax.dot_general(i8 * off, ones81, (((1,), (0,)), ((), ())),
                              precision=lax.Precision.HIGHEST,
                              preferred_element_type=f32)
    cnt_col = lax.dot_general(i8 * counts, ones81, (((1,), (0,)), ((), ())),
                              precision=lax.Precision.HIGHEST,
                              preferred_element_type=f32)
    bs_col = jnp.floor(off_col / bmf)
    nb_col = jnp.floor((cnt_col + (bmf - 1.0)) / bmf)
    bi = lax.broadcasted_iota(jnp.int32, (E, nbpad), 1).astype(f32)
    belongs = ((bi >= bs_col) & (bi < bs_col + nb_col)).astype(f32)
    erow = lax.broadcasted_iota(jnp.int32, (E, nbpad), 0).astype(f32)
    bexp = jnp.sum(erow * belongs, axis=0, keepdims=True)
    assigned = jnp.sum(belongs, axis=0, keepdims=True)
    bexp = jnp.where(assigned > 0, bexp, jnp.float32(E - 1))
    bexp_ref[0, :] = bexp.astype(jnp.int32)[0]

    nonempty = (nb_col > 0).astype(f32)
    isf = jnp.sum(((bi == bs_col).astype(f32)) * nonempty, axis=0,
                  keepdims=True)
    ut128 = (lax.broadcasted_iota(jnp.int32, (nbpad, nbpad), 0)
             <= lax.broadcasted_iota(jnp.int32, (nbpad, nbpad), 1)).astype(f32)
    cum = lax.dot_general(isf, ut128, (((1,), (0,)), ((), ())),
                          precision=lax.Precision.HIGHEST,
                          preferred_element_type=f32)
    parity = (cum.astype(jnp.int32) - 1) & 1
    cand = jnp.where((bs_col > bi) & (nb_col > 0), erow, jnp.float32(E))
    nexte = jnp.min(cand, axis=0, keepdims=True)
    bexp_ref[1, :] = isf.astype(jnp.int32)[0]
    bexp_ref[2, :] = parity[0]
    bexp_ref[3, :] = nexte.astype(jnp.int32)[0]


def _dispatch_body(x_hbm, pos_hbm, xs_hbm, rows0_v, rows1_v, idx_v,
                   semL0, semL1, semS):
    T, H = x_hbm.shape
    ch = rows0_v.shape[0]
    tpw = T // NW
    wid = lax.axis_index("s") * NC + lax.axis_index("c")
    base = wid * tpw
    nk = tpw // ch
    rows = (rows0_v, rows1_v)
    semsl = (semL0, semL1)
    pltpu.sync_copy(pos_hbm.at[pl.ds(base, tpw)], idx_v.at[0])
    pltpu.sync_copy(pos_hbm.at[pl.ds(T + base, tpw)], idx_v.at[1])
    loads = [None] * nk
    loads[0] = pltpu.async_copy(x_hbm.at[pl.ds(base, ch), :], rows[0], semsl[0])
    for k in range(nk):
        cur = k % 2
        if k + 1 < nk:
            loads[k + 1] = pltpu.async_copy(
                x_hbm.at[pl.ds(base + (k + 1) * ch, ch), :],
                rows[(k + 1) % 2], semsl[(k + 1) % 2])
        loads[k].wait()
        i0 = idx_v[0, pl.ds(k * ch, ch)]
        i1 = idx_v[1, pl.ds(k * ch, ch)]
        d0 = pltpu.async_copy(rows[cur], xs_hbm.at[i0], semS)
        d1 = pltpu.async_copy(rows[cur], xs_hbm.at[i1], semS)
        d0.wait()
        d1.wait()


def _combine_body(ys_hbm, pos_hbm, wbc_hbm, out_hbm,
                  g0a_v, g0b_v, g1a_v, g1b_v, oa_v, ob_v,
                  i0a_v, i0b_v, i1a_v, i1b_v, wb_v,
                  semGa, semGb, semOa, semOb):
    T, H = out_hbm.shape
    ct = g0a_v.shape[0]
    tpw = T // NW
    wid = lax.axis_index("s") * NC + lax.axis_index("c")
    base = wid * tpw
    nk = tpw // ct
    g0 = (g0a_v, g0b_v)
    g1 = (g1a_v, g1b_v)
    ov = (oa_v, ob_v)
    pltpu.sync_copy(wbc_hbm.at[pl.ds(base, tpw), :], wb_v.at[0])
    pltpu.sync_copy(wbc_hbm.at[pl.ds(T + base, tpw), :], wb_v.at[1])

    semg = (semGa, semGb)
    semo = (semOa, semOb)
    i0r = (i0a_v, i0b_v)
    i1r = (i1a_v, i1b_v)

    def gathers(k):
        s = k % 2
        pltpu.sync_copy(pos_hbm.at[pl.ds(base + k * ct, ct)], i0r[s])
        pltpu.sync_copy(pos_hbm.at[pl.ds(T + base + k * ct, ct)], i1r[s])
        return (pltpu.async_copy(ys_hbm.at[i0r[s]], g0[s], semg[s]),
                pltpu.async_copy(ys_hbm.at[i1r[s]], g1[s], semg[s]))

    pend = gathers(0)
    stores = [None] * nk
    for k in range(nk):
        s = k % 2
        nxt = None
        if k + 1 < nk:
            nxt = gathers(k + 1)
        pend[0].wait()
        pend[1].wait()
        pend = nxt
        if k >= 2:
            stores[k - 2].wait()
        for i in range(ct):
            wb0 = wb_v[0, k * ct + i]
            wb1 = wb_v[1, k * ct + i]

            @plsc.parallel_loop(0, H, step=16, unroll=8)
            def _mul(c):
                sl = pl.ds(c, 16)
                ov[s][i, sl] = g0[s][i, sl] * wb0 + g1[s][i, sl] * wb1
        stores[k] = pltpu.async_copy(
            ov[s], out_hbm.at[pl.ds(base + k * ct, ct), :], semo[s])
    stores[nk - 2].wait()
    stores[nk - 1].wait()


def _mm_body(meta_ref, xs_ref, gu_hbm, dn_hbm, ys_ref,
             gu0, gu1, dn0, dn1, sg0, sg1, sd0, sd1):
    nbpad = 128
    I = dn0.shape[0]
    b = pl.program_id(0)
    e = meta_ref[b]
    isf = meta_ref[nbpad + b]
    p = meta_ref[2 * nbpad + b]
    ne = meta_ref[3 * nbpad + b]
    gubuf = (gu0, gu1)
    dnbuf = (dn0, dn1)
    semg = (sg0, sg1)
    semd = (sd0, sd1)

    def issue(slot, ex):
        pltpu.make_async_copy(gu_hbm.at[ex], gubuf[slot], semg[slot]).start()
        pltpu.make_async_copy(dn_hbm.at[ex], dnbuf[slot], semd[slot]).start()

    def waitw(slot, ex):
        pltpu.make_async_copy(gu_hbm.at[ex], gubuf[slot], semg[slot]).wait()
        pltpu.make_async_copy(dn_hbm.at[ex], dnbuf[slot], semd[slot]).wait()

    @pl.when(b == 0)
    def _cold():
        issue(0, e)

    for slot in range(2):
        @pl.when((isf == 1) & (p == slot))
        def _first(slot=slot):
            waitw(slot, e)

        @pl.when((isf == 1) & (ne < E) & (p == slot))
        def _issue_next(slot=slot):
            issue(1 - slot, ne)

        @pl.when(p == slot)
        def _compute(slot=slot):
            h1 = lax.dot_general(xs_ref[...], gubuf[slot][...],
                                 (((1,), (0,)), ((), ())),
                                 preferred_element_type=jnp.float32)
            g = h1[:, :I]
            u = h1[:, I:]
            act = g * jax.nn.sigmoid(g) * u
            ys_ref[...] = lax.dot_general(act, dnbuf[slot][...],
                                          (((1,), (0,)), ((), ())),
                                          preferred_element_type=jnp.float32)


def kernel(hidden_states, gate_weight, gate_up_proj, down_proj):
    b, s, h = hidden_states.shape
    e, _, i2 = gate_up_proj.shape
    i = i2 // 2
    T = b * s
    nb = (2 * T) // BM + e - 1
    pad = nb * BM
    nbpad = 128

    x = hidden_states.reshape(T, h)

    pos2, wbc3, meta_rows = pl.pallas_call(
        _router_body,
        out_shape=(
            jax.ShapeDtypeStruct((2, T), jnp.int32),
            jax.ShapeDtypeStruct((2, T, 16), jnp.float32),
            jax.ShapeDtypeStruct((4, nbpad), jnp.int32),
        ),
    )(x, gate_weight)
    pos = pos2.reshape(2 * T)
    wbc = wbc3.reshape(2 * T, 16)
    meta = meta_rows.reshape(4 * nbpad)

    ch = 16
    tpw = T // NW
    mesh = plsc.VectorSubcoreMesh(core_axis_name="c", subcore_axis_name="s")
    xs = pl.kernel(
        _dispatch_body,
        out_type=jax.ShapeDtypeStruct((pad, h), jnp.float32),
        mesh=mesh,
        scratch_types=[
            pltpu.VMEM((ch, h), jnp.float32),
            pltpu.VMEM((ch, h), jnp.float32),
            pltpu.VMEM((2, tpw), jnp.int32),
            pltpu.SemaphoreType.DMA,
            pltpu.SemaphoreType.DMA,
            pltpu.SemaphoreType.DMA,
        ],
    )(x, pos)

    grid_spec = pltpu.PrefetchScalarGridSpec(
        num_scalar_prefetch=1,
        grid=(nb,),
        in_specs=[
            pl.BlockSpec((BM, h), lambda b_, be: (b_, 0)),
            pl.BlockSpec(memory_space=pl.ANY),
            pl.BlockSpec(memory_space=pl.ANY),
        ],
        out_specs=pl.BlockSpec((BM, h), lambda b_, be: (b_, 0)),
        scratch_shapes=[
            pltpu.VMEM((h, i2), jnp.float32),
            pltpu.VMEM((h, i2), jnp.float32),
            pltpu.VMEM((i, h), jnp.float32),
            pltpu.VMEM((i, h), jnp.float32),
            pltpu.SemaphoreType.DMA,
            pltpu.SemaphoreType.DMA,
            pltpu.SemaphoreType.DMA,
            pltpu.SemaphoreType.DMA,
        ],
    )
    ys = pl.pallas_call(
        _mm_body,
        grid_spec=grid_spec,
        out_shape=jax.ShapeDtypeStruct((pad, h), jnp.float32),
    )(meta, xs, gate_up_proj, down_proj)

    ct = 8
    out = pl.kernel(
        _combine_body,
        out_type=jax.ShapeDtypeStruct((T, h), jnp.float32),
        mesh=mesh,
        scratch_types=[
            pltpu.VMEM((ct, h), jnp.float32),
            pltpu.VMEM((ct, h), jnp.float32),
            pltpu.VMEM((ct, h), jnp.float32),
            pltpu.VMEM((ct, h), jnp.float32),
            pltpu.VMEM((ct, h), jnp.float32),
            pltpu.VMEM((ct, h), jnp.float32),
            pltpu.VMEM((ct,), jnp.int32),
            pltpu.VMEM((ct,), jnp.int32),
            pltpu.VMEM((ct,), jnp.int32),
            pltpu.VMEM((ct,), jnp.int32),
            pltpu.VMEM((2, tpw, 16), jnp.float32),
            pltpu.SemaphoreType.DMA,
            pltpu.SemaphoreType.DMA,
            pltpu.SemaphoreType.DMA,
            pltpu.SemaphoreType.DMA,
        ],
    )(ys, pos, wbc)

    return out.reshape(b, s, h)
